# baseline (device time: 298518 ns/iter reference)
import jax
import jax.numpy as jnp
from jax import lax
from jax.experimental import pallas as pl
from jax.experimental.pallas import tpu as pltpu

N_DEV = 32
B, SQ, D_MODEL = 2, 512, 768
H_LOC, DH = 8, 64
WINDOW = 128
QBLK = 128
KBAND = 384
ZROWS, YROWS, XROWS = 128, 32, 16


def kernel(x, Wq, K_ext, V_ext, Wo):
    def body(x_ref, wq_ref, k_any, v_any, wo_ref, out_ref,
             q_ref, k_ref, v_ref, ctx_ref, sbz, rcz, sby, rcy, sbx, rcx,
             kv_sems, send_sems, recv_sems):
        p = lax.axis_index("i")
        mz = p // 8
        r = lax.rem(p, 8)
        my_y = r // 2
        mx = lax.rem(lax.rem(r, 2) + lax.rem(my_y, 2), 2)

        def pos(xx, yy, zz):
            return 8 * zz + 2 * yy + lax.rem(xx + lax.rem(yy, 2), 2)

        z_next = pos(mx, my_y, lax.rem(mz + 1, 4))
        z_prev = pos(mx, my_y, lax.rem(mz + 3, 4))
        y_next = pos(mx, lax.rem(my_y + 1, 4), mz)
        y_prev = pos(mx, lax.rem(my_y + 3, 4), mz)
        x_part = pos(1 - mx, my_y, mz)

        k_copy = pltpu.make_async_copy(
            k_any.at[:, :, pl.ds(p * H_LOC, H_LOC), :], k_ref, kv_sems.at[0]
        )
        v_copy = pltpu.make_async_copy(
            v_any.at[:, :, pl.ds(p * H_LOC, H_LOC), :], v_ref, kv_sems.at[1]
        )
        k_copy.start()
        v_copy.start()

        partners = (z_next, z_prev, y_next, y_prev, x_part)
        barrier_sem = pltpu.get_barrier_semaphore()
        for nbr in partners:
            pl.semaphore_signal(
                barrier_sem, inc=1,
                device_id=(nbr,), device_id_type=pl.DeviceIdType.MESH,
            )
        pl.semaphore_wait(barrier_sem, len(partners))

        x2 = x_ref[...].reshape(B * SQ, D_MODEL)
        q_ref[...] = jnp.dot(
            x2, wq_ref[...], preferred_element_type=jnp.float32
        )
        k_copy.wait()
        v_copy.wait()

        def compute_group(g):
            kb = jnp.clip((g - 1) * QBLK, 0, SQ - KBAND)
            qi = g * QBLK + lax.broadcasted_iota(jnp.int32, (QBLK, KBAND), 0)
            kj = kb + lax.broadcasted_iota(jnp.int32, (QBLK, KBAND), 1)
            mask = jnp.abs(qi - kj) <= WINDOW
            for b in range(B):
                for h in range(H_LOC):
                    qh = q_ref[pl.ds(b * SQ + g * QBLK, QBLK),
                               h * DH:(h + 1) * DH]
                    kh = k_ref[b, pl.ds(kb, KBAND), h, :]
                    s = lax.dot_general(
                        qh, kh, (((1,), (1,)), ((), ())),
                        preferred_element_type=jnp.float32,
                    ) * 0.125
                    s = jnp.where(mask, s, -1e9)
                    m = jnp.max(s, axis=1, keepdims=True)
                    w = jnp.exp(s - m)
                    w = w / jnp.sum(w, axis=1, keepdims=True)
                    ctx_ref[b, pl.ds(g * QBLK, QBLK),
                            h * DH:(h + 1) * DH] = jnp.dot(
                        w, v_ref[b, pl.ds(kb, KBAND), h, :],
                        preferred_element_type=jnp.float32,
                    )
            for b in range(B):
                out_ref[b, pl.ds(g * QBLK, QBLK), :] = jnp.dot(
                    ctx_ref[b, pl.ds(g * QBLK, QBLK), :], wo_ref[...],
                    preferred_element_type=jnp.float32,
                )

        def make_rdma(src, dst, sem_idx, target):
            return pltpu.make_async_remote_copy(
                src_ref=src, dst_ref=dst,
                send_sem=send_sems.at[sem_idx],
                recv_sem=recv_sems.at[sem_idx],
                device_id=(target,),
                device_id_type=pl.DeviceIdType.MESH,
            )

        ops = []
        for s in range(3):
            g = lax.rem(mz - s + 4, 4)
            compute_group(g)
            if s > 0:
                ops[s - 1].wait_recv()
            loc = out_ref[:, pl.ds(g * ZROWS, ZROWS), :]
            sbz[s] = loc if s == 0 else rcz[s - 1] + loc
            op = make_rdma(sbz.at[s], rcz.at[s], s, z_next)
            op.start()
            ops.append(op)
            if s > 0:
                ops[s - 1].wait_send()
        own_z = lax.rem(mz + 1, 4)
        compute_group(own_z)
        ops[2].wait_recv()
        zrow = own_z * ZROWS
        out_ref[:, pl.ds(zrow, ZROWS), :] = (
            out_ref[:, pl.ds(zrow, ZROWS), :] + rcz[2]
        )
        ops[2].wait_send()

        def rdma_step(src, dst, sem_idx, target):
            op = make_rdma(src, dst, sem_idx, target)
            op.start()
            op.wait()

        for s in range(3):
            gy = lax.rem(my_y - s + 4, 4)
            loc = out_ref[:, pl.ds(zrow + gy * YROWS, YROWS), :]
            sby[s] = loc if s == 0 else rcy[s - 1] + loc
            rdma_step(sby.at[s], rcy.at[s], 3 + s, y_next)
        own_y = lax.rem(my_y + 1, 4)
        yrow = zrow + own_y * YROWS
        out_ref[:, pl.ds(yrow, YROWS), :] = (
            out_ref[:, pl.ds(yrow, YROWS), :] + rcy[2]
        )

        gx = 1 - mx
        sbx[0] = out_ref[:, pl.ds(yrow + gx * XROWS, XROWS), :]
        rdma_step(sbx.at[0], rcx.at[0], 6, x_part)
        own_row = yrow + mx * XROWS
        out_ref[:, pl.ds(own_row, XROWS), :] = (
            out_ref[:, pl.ds(own_row, XROWS), :] + rcx[0]
        )

        rdma_step(
            out_ref.at[:, pl.ds(own_row, XROWS), :],
            out_ref.at[:, pl.ds(own_row, XROWS), :],
            7, x_part,
        )
        for t in range(3):
            gy = lax.rem(my_y + 1 - t + 4, 4)
            row0 = zrow + gy * YROWS
            rdma_step(
                out_ref.at[:, pl.ds(row0, YROWS), :],
                out_ref.at[:, pl.ds(row0, YROWS), :],
                8 + t, y_next,
            )
        for t in range(3):
            g = lax.rem(mz + 1 - t + 4, 4)
            row0 = g * ZROWS
            rdma_step(
                out_ref.at[:, pl.ds(row0, ZROWS), :],
                out_ref.at[:, pl.ds(row0, ZROWS), :],
                11 + t, z_next,
            )

    return pl.pallas_call(
        body,
        out_shape=jax.ShapeDtypeStruct((B, SQ, D_MODEL), jnp.float32),
        in_specs=[
            pl.BlockSpec(memory_space=pltpu.VMEM),
            pl.BlockSpec(memory_space=pltpu.VMEM),
            pl.BlockSpec(memory_space=pltpu.MemorySpace.HBM),
            pl.BlockSpec(memory_space=pltpu.MemorySpace.HBM),
            pl.BlockSpec(memory_space=pltpu.VMEM),
        ],
        out_specs=pl.BlockSpec(memory_space=pltpu.VMEM),
        scratch_shapes=[
            pltpu.VMEM((B * SQ, H_LOC * DH), jnp.float32),
            pltpu.VMEM((B, SQ, H_LOC, DH), jnp.float32),
            pltpu.VMEM((B, SQ, H_LOC, DH), jnp.float32),
            pltpu.VMEM((B, SQ, H_LOC * DH), jnp.float32),
            pltpu.VMEM((3, B, ZROWS, D_MODEL), jnp.float32),
            pltpu.VMEM((3, B, ZROWS, D_MODEL), jnp.float32),
            pltpu.VMEM((3, B, YROWS, D_MODEL), jnp.float32),
            pltpu.VMEM((3, B, YROWS, D_MODEL), jnp.float32),
            pltpu.VMEM((1, B, XROWS, D_MODEL), jnp.float32),
            pltpu.VMEM((1, B, XROWS, D_MODEL), jnp.float32),
            pltpu.SemaphoreType.DMA((2,)),
            pltpu.SemaphoreType.DMA((14,)),
            pltpu.SemaphoreType.DMA((14,)),
        ],
        compiler_params=pltpu.CompilerParams(collective_id=0),
    )(x, Wq, K_ext, V_ext, Wo)


# device time: 215035 ns/iter; 1.3882x vs baseline; 1.3882x over previous
import jax
import jax.numpy as jnp
from jax import lax
from jax.experimental import pallas as pl
from jax.experimental.pallas import tpu as pltpu

N_DEV = 32
B, SQ, D_MODEL = 2, 512, 768
H_LOC, DH = 8, 64
WINDOW = 128
HALF = D_MODEL // 2
ZROWS, YROWS, XROWS = 128, 32, 16


def kernel(x, Wq, K_ext, V_ext, Wo):
    my = lax.axis_index("i")
    k_loc = lax.dynamic_slice_in_dim(K_ext, my * H_LOC, H_LOC, axis=2)
    v_loc = lax.dynamic_slice_in_dim(V_ext, my * H_LOC, H_LOC, axis=2)

    def body(x_ref, wq_ref, k_ref, v_ref, wo_ref, out_ref,
             ctx_ref,
             sbzA, rczA, sbyA, rcyA, sbxA, rcxA,
             sbzB, rczB, sbyB, rcyB, sbxB, rcxB,
             ssemA, rsemA, ssemB, rsemB):
        p = lax.axis_index("i")
        mz = p // 8
        r = lax.rem(p, 8)
        my_y = r // 2
        mx = lax.rem(lax.rem(r, 2) + lax.rem(my_y, 2), 2)

        def pos(xx, yy, zz):
            return 8 * zz + 2 * yy + lax.rem(xx + lax.rem(yy, 2), 2)

        z_next = pos(mx, my_y, lax.rem(mz + 1, 4))
        z_prev = pos(mx, my_y, lax.rem(mz + 3, 4))
        y_next = pos(mx, lax.rem(my_y + 1, 4), mz)
        y_prev = pos(mx, lax.rem(my_y + 3, 4), mz)
        x_part = pos(1 - mx, my_y, mz)

        partners = (z_next, z_prev, y_next, y_prev, x_part)
        barrier_sem = pltpu.get_barrier_semaphore()
        for nbr in partners:
            pl.semaphore_signal(
                barrier_sem, inc=1,
                device_id=(nbr,), device_id_type=pl.DeviceIdType.MESH,
            )
        pl.semaphore_wait(barrier_sem, len(partners))

        x2 = x_ref[...].reshape(B * SQ, D_MODEL)
        q = jnp.dot(x2, wq_ref[...], preferred_element_type=jnp.float32)

        qi = lax.broadcasted_iota(jnp.int32, (SQ, SQ), 0)
        ki = lax.broadcasted_iota(jnp.int32, (SQ, SQ), 1)
        mask = jnp.abs(qi - ki) <= WINDOW

        for b in range(B):
            for h in range(H_LOC):
                qh = q[b * SQ:(b + 1) * SQ, h * DH:(h + 1) * DH]
                kh = k_ref[b, :, h, :]
                s = lax.dot_general(
                    qh, kh, (((1,), (1,)), ((), ())),
                    preferred_element_type=jnp.float32,
                ) * 0.125
                s = jnp.where(mask, s, -1e9)
                m = jnp.max(s, axis=1, keepdims=True)
                w = jnp.exp(s - m)
                w = w / jnp.sum(w, axis=1, keepdims=True)
                ctx_ref[b, :, h * DH:(h + 1) * DH] = jnp.dot(
                    w, v_ref[b, :, h, :], preferred_element_type=jnp.float32
                )
        for b in range(B):
            out_ref[b, :, :] = jnp.dot(
                ctx_ref[b, :, :], wo_ref[...],
                preferred_element_type=jnp.float32,
            )

        tracks = [
            dict(sig=1, zn=z_next, yn=y_next, c0=0,
                 sbz=sbzA, rcz=rczA, sby=sbyA, rcy=rcyA,
                 sbx=sbxA, rcx=rcxA, ss=ssemA, rs=rsemA),
            dict(sig=-1, zn=z_prev, yn=y_prev, c0=HALF,
                 sbz=sbzB, rcz=rczB, sby=sbyB, rcy=rcyB,
                 sbx=sbxB, rcx=rcxB, ss=ssemB, rs=rsemB),
        ]
        for tr in tracks:
            tr['own_z'] = lax.rem(mz + tr['sig'] + 4, 4)
            tr['zrow'] = tr['own_z'] * ZROWS
            tr['own_y'] = lax.rem(my_y + tr['sig'] + 4, 4)

        def start_rdma(tr, src, dst, sem_idx, target):
            op = pltpu.make_async_remote_copy(
                src_ref=src, dst_ref=dst,
                send_sem=tr['ss'].at[sem_idx],
                recv_sem=tr['rs'].at[sem_idx],
                device_id=(target,),
                device_id_type=pl.DeviceIdType.MESH,
            )
            op.start()
            return op

        def csl(c0):
            return slice(c0, c0 + HALF)

        for s in range(3):
            ops = []
            for tr in tracks:
                g = lax.rem(mz - tr['sig'] * s + 8, 4)
                loc = out_ref[:, pl.ds(g * ZROWS, ZROWS), csl(tr['c0'])]
                tr['sbz'][s] = loc if s == 0 else tr['rcz'][s - 1] + loc
                ops.append(start_rdma(
                    tr, tr['sbz'].at[s], tr['rcz'].at[s], s, tr['zn']))
            for op in ops:
                op.wait()
        for tr in tracks:
            zrow, c = tr['zrow'], csl(tr['c0'])
            out_ref[:, pl.ds(zrow, ZROWS), c] = (
                out_ref[:, pl.ds(zrow, ZROWS), c] + tr['rcz'][2]
            )

        for s in range(3):
            ops = []
            for tr in tracks:
                gy = lax.rem(my_y - tr['sig'] * s + 8, 4)
                row0 = tr['zrow'] + gy * YROWS
                loc = out_ref[:, pl.ds(row0, YROWS), csl(tr['c0'])]
                tr['sby'][s] = loc if s == 0 else tr['rcy'][s - 1] + loc
                ops.append(start_rdma(
                    tr, tr['sby'].at[s], tr['rcy'].at[s], 3 + s, tr['yn']))
            for op in ops:
                op.wait()
        for tr in tracks:
            tr['yrow'] = tr['zrow'] + tr['own_y'] * YROWS
            yrow, c = tr['yrow'], csl(tr['c0'])
            out_ref[:, pl.ds(yrow, YROWS), c] = (
                out_ref[:, pl.ds(yrow, YROWS), c] + tr['rcy'][2]
            )

        ops = []
        gx = 1 - mx
        for tr in tracks:
            tr['sbx'][0] = out_ref[
                :, pl.ds(tr['yrow'] + gx * XROWS, XROWS), csl(tr['c0'])]
            ops.append(start_rdma(
                tr, tr['sbx'].at[0], tr['rcx'].at[0], 6, x_part))
        for op in ops:
            op.wait()
        for tr in tracks:
            tr['own_row'] = tr['yrow'] + mx * XROWS
            row0, c = tr['own_row'], csl(tr['c0'])
            out_ref[:, pl.ds(row0, XROWS), c] = (
                out_ref[:, pl.ds(row0, XROWS), c] + tr['rcx'][0]
            )

        ops = []
        for tr in tracks:
            sl = out_ref.at[:, pl.ds(tr['own_row'], XROWS),
                            pl.ds(tr['c0'], HALF)]
            ops.append(start_rdma(tr, sl, sl, 7, x_part))
        for op in ops:
            op.wait()
        for t in range(3):
            ops = []
            for tr in tracks:
                gy = lax.rem(my_y + tr['sig'] - tr['sig'] * t + 8, 4)
                row0 = tr['zrow'] + gy * YROWS
                sl = out_ref.at[:, pl.ds(row0, YROWS), pl.ds(tr['c0'], HALF)]
                ops.append(start_rdma(tr, sl, sl, 8 + t, tr['yn']))
            for op in ops:
                op.wait()
        for t in range(3):
            ops = []
            for tr in tracks:
                g = lax.rem(mz + tr['sig'] - tr['sig'] * t + 8, 4)
                row0 = g * ZROWS
                sl = out_ref.at[:, pl.ds(row0, ZROWS), pl.ds(tr['c0'], HALF)]
                ops.append(start_rdma(tr, sl, sl, 11 + t, tr['zn']))
            for op in ops:
                op.wait()

    return pl.pallas_call(
        body,
        out_shape=jax.ShapeDtypeStruct((B, SQ, D_MODEL), jnp.float32),
        in_specs=[pl.BlockSpec(memory_space=pltpu.VMEM)] * 5,
        out_specs=pl.BlockSpec(memory_space=pltpu.VMEM),
        scratch_shapes=[
            pltpu.VMEM((B, SQ, H_LOC * DH), jnp.float32),
            pltpu.VMEM((3, B, ZROWS, HALF), jnp.float32),
            pltpu.VMEM((3, B, ZROWS, HALF), jnp.float32),
            pltpu.VMEM((3, B, YROWS, HALF), jnp.float32),
            pltpu.VMEM((3, B, YROWS, HALF), jnp.float32),
            pltpu.VMEM((1, B, XROWS, HALF), jnp.float32),
            pltpu.VMEM((1, B, XROWS, HALF), jnp.float32),
            pltpu.VMEM((3, B, ZROWS, HALF), jnp.float32),
            pltpu.VMEM((3, B, ZROWS, HALF), jnp.float32),
            pltpu.VMEM((3, B, YROWS, HALF), jnp.float32),
            pltpu.VMEM((3, B, YROWS, HALF), jnp.float32),
            pltpu.VMEM((1, B, XROWS, HALF), jnp.float32),
            pltpu.VMEM((1, B, XROWS, HALF), jnp.float32),
            pltpu.SemaphoreType.DMA((14,)),
            pltpu.SemaphoreType.DMA((14,)),
            pltpu.SemaphoreType.DMA((14,)),
            pltpu.SemaphoreType.DMA((14,)),
        ],
        compiler_params=pltpu.CompilerParams(collective_id=0),
    )(x, Wq, k_loc, v_loc, Wo)
